# baseline (speedup 1.0000x reference)
import functools

import jax
import jax.numpy as jnp
from jax import lax
from jax.experimental import pallas as pl
from jax.experimental.pallas import tpu as pltpu
from jax.experimental.pallas import tpu_sc as plsc

ENT_DIM = 64
NB = 128


def _sc_gather(table2, idx):
    n = idx.shape[0]
    NW = 32
    CHUNK = n // NW
    mesh = plsc.VectorSubcoreMesh(core_axis_name="c", subcore_axis_name="s")

    @functools.partial(
        pl.kernel,
        mesh=mesh,
        out_type=jax.ShapeDtypeStruct((n, 2 * ENT_DIM), jnp.float32),
        scratch_types=[
            pltpu.VMEM((CHUNK,), jnp.int32),
            pltpu.VMEM((CHUNK, 2 * ENT_DIM), jnp.float32),
            pltpu.SemaphoreType.DMA,
        ],
    )
    def k(tab_hbm, idx_hbm, out_hbm, idx_v, rows_v, sem):
        wid = lax.axis_index("s") * 2 + lax.axis_index("c")
        base = wid * CHUNK
        pltpu.sync_copy(idx_hbm.at[pl.ds(base, CHUNK)], idx_v)
        pltpu.async_copy(tab_hbm.at[idx_v], rows_v, sem).wait()
        pltpu.sync_copy(rows_v, out_hbm.at[pl.ds(base, CHUNK)])

    return k(table2, idx)


def _sel_half(x2, par):
    return x2[:, :ENT_DIM] * (1.0 - par) + x2[:, ENT_DIM:] * par


def _tc_body(rpos_s, rneg_s, proj_ref, rel_ref,
             hp_ref, tp_ref, hn_ref, tn_ref, par_ref,
             ph_ref, pt_ref, nh_ref, nt_ref, pos_ref, neg_ref,
             pp_scr, pn_scr, rp_scr, rn_scr):
    i = pl.program_id(0)

    def gather_body(n, carry):
        rp = rpos_s[i * NB + n]
        rn = rneg_s[i * NB + n]
        pp_scr[n] = proj_ref[rp]
        pn_scr[n] = proj_ref[rn]
        rp_scr[n] = rel_ref[rp]
        rn_scr[n] = rel_ref[rn]
        return carry

    lax.fori_loop(0, NB, gather_body, 0)

    par = par_ref[0]
    hp = _sel_half(hp_ref[...], par[0][:, None])
    tp = _sel_half(tp_ref[...], par[1][:, None])
    hn = _sel_half(hn_ref[...], par[2][:, None])
    tn = _sel_half(tn_ref[...], par[3][:, None])

    def matvec(p, e):
        e2 = jnp.concatenate([e, e], axis=1)
        prod = p * e2[:, None, :]
        lo = jnp.sum(prod[:, :, :ENT_DIM], axis=2)
        hi = jnp.sum(prod[:, :, ENT_DIM:], axis=2)
        return jnp.stack([lo, hi], axis=2).reshape(NB, ENT_DIM)

    ppv = pp_scr[...]
    pnv = pn_scr[...]
    ph = matvec(ppv, hp)
    pt = matvec(ppv, tp)
    nh = matvec(pnv, hn)
    nt = matvec(pnv, tn)
    ph_ref[...] = ph
    pt_ref[...] = pt
    nh_ref[...] = nh
    nt_ref[...] = nt
    pos_ref[...] = jnp.sum(jnp.abs(ph + rp_scr[...] - pt), axis=1)
    neg_ref[...] = jnp.sum(jnp.abs(nh + rn_scr[...] - nt), axis=1)


def _tc_compute(pos_r, neg_r, proj3, rel_w, ent_g2, par4, interpret=False):
    b = pos_r.shape[0]
    grid = (b // NB,)
    nblk = b // NB
    pair_specs = [
        pl.BlockSpec((NB, 2 * ENT_DIM),
                     functools.partial(lambda q, i, *_: (q * nblk + i, 0), q))
        for q in range(4)
    ]
    vec_spec = pl.BlockSpec((NB, ENT_DIM), lambda i, *_: (i, 0))
    out_shapes = (
        jax.ShapeDtypeStruct((b, ENT_DIM), jnp.float32),
        jax.ShapeDtypeStruct((b, ENT_DIM), jnp.float32),
        jax.ShapeDtypeStruct((b, ENT_DIM), jnp.float32),
        jax.ShapeDtypeStruct((b, ENT_DIM), jnp.float32),
        jax.ShapeDtypeStruct((b,), jnp.float32),
        jax.ShapeDtypeStruct((b,), jnp.float32),
    )
    return pl.pallas_call(
        _tc_body,
        grid_spec=pltpu.PrefetchScalarGridSpec(
            num_scalar_prefetch=2,
            grid=grid,
            in_specs=[
                pl.BlockSpec((proj3.shape[0], ENT_DIM // 2, 2 * ENT_DIM),
                             lambda i, *_: (0, 0, 0)),
                pl.BlockSpec((rel_w.shape[0], ENT_DIM),
                             lambda i, *_: (0, 0)),
                *pair_specs,
                pl.BlockSpec((1, 4, NB), lambda i, *_: (i, 0, 0)),
            ],
            out_specs=[
                vec_spec, vec_spec, vec_spec, vec_spec,
                pl.BlockSpec((NB,), lambda i, *_: (i,)),
                pl.BlockSpec((NB,), lambda i, *_: (i,)),
            ],
            scratch_shapes=[
                pltpu.VMEM((NB, ENT_DIM // 2, 2 * ENT_DIM), jnp.float32),
                pltpu.VMEM((NB, ENT_DIM // 2, 2 * ENT_DIM), jnp.float32),
                pltpu.VMEM((NB, ENT_DIM), jnp.float32),
                pltpu.VMEM((NB, ENT_DIM), jnp.float32),
            ],
        ),
        out_shape=out_shapes,
        interpret=interpret,
    )(pos_r, neg_r, proj3, rel_w, ent_g2, ent_g2, ent_g2, ent_g2, par4)


def kernel(pos_h, pos_t, pos_r, neg_h, neg_t, neg_r, ent_w, rel_w, proj_w):
    b = pos_h.shape[0]
    ent_idx = jnp.concatenate(
        [pos_h, pos_t, neg_h, neg_t]).astype(jnp.int32)
    ent2 = ent_w.reshape(ent_w.shape[0] // 2, 2 * ENT_DIM)
    ent_g2 = _sc_gather(ent2, ent_idx // 2)
    par4 = (ent_idx & 1).astype(jnp.float32).reshape(
        4, b // NB, NB).transpose(1, 0, 2)
    proj3 = proj_w.reshape(proj_w.shape[0], ENT_DIM // 2, 2 * ENT_DIM)
    ph, pt, nh, nt, pos, neg = _tc_compute(
        pos_r.astype(jnp.int32), neg_r.astype(jnp.int32),
        proj3, rel_w, ent_g2, par4)
    return (pos, neg, ph, pt, nh, nt)


# speedup vs baseline: 1.0106x; 1.0106x over previous
import functools

import jax
import jax.numpy as jnp
from jax import lax
from jax.experimental import pallas as pl
from jax.experimental.pallas import tpu as pltpu
from jax.experimental.pallas import tpu_sc as plsc

ENT_DIM = 64
NB = 128


def _sc_gather(table2, idx):
    n = idx.shape[0]
    NW = 32
    CHUNK = n // NW
    mesh = plsc.VectorSubcoreMesh(core_axis_name="c", subcore_axis_name="s")

    @functools.partial(
        pl.kernel,
        mesh=mesh,
        out_type=jax.ShapeDtypeStruct((n, 2 * ENT_DIM), jnp.float32),
        scratch_types=[
            pltpu.VMEM((CHUNK,), jnp.int32),
            pltpu.VMEM((CHUNK, 2 * ENT_DIM), jnp.float32),
            pltpu.SemaphoreType.DMA,
        ],
    )
    def k(tab_hbm, idx_hbm, out_hbm, idx_v, rows_v, sem):
        wid = lax.axis_index("s") * 2 + lax.axis_index("c")
        base = wid * CHUNK
        pltpu.sync_copy(idx_hbm.at[pl.ds(base, CHUNK)], idx_v)
        pltpu.async_copy(tab_hbm.at[idx_v], rows_v, sem).wait()
        pltpu.sync_copy(rows_v, out_hbm.at[pl.ds(base, CHUNK)])

    return k(table2, idx)


def _sel_half(x2, par):
    return x2[:, :ENT_DIM] * (1.0 - par) + x2[:, ENT_DIM:] * par


def _tc_body(rpos_s, rneg_s, proj_ref,
             hp_ref, tp_ref, hn_ref, tn_ref, rp_ref, rn_ref, par_ref,
             ph_ref, pt_ref, nh_ref, nt_ref, pos_ref, neg_ref,
             pp_scr, pn_scr):
    i = pl.program_id(0)

    def gather_body(n, carry):
        pp_scr[n] = proj_ref[rpos_s[i * NB + n]]
        pn_scr[n] = proj_ref[rneg_s[i * NB + n]]
        return carry

    lax.fori_loop(0, NB, gather_body, 0)

    par = par_ref[0]
    hp = _sel_half(hp_ref[...], par[0][:, None])
    tp = _sel_half(tp_ref[...], par[1][:, None])
    hn = _sel_half(hn_ref[...], par[2][:, None])
    tn = _sel_half(tn_ref[...], par[3][:, None])
    rp = _sel_half(rp_ref[...], par[4][:, None])
    rn = _sel_half(rn_ref[...], par[5][:, None])

    def matvec(p, e):
        e2 = jnp.concatenate([e, e], axis=1)
        prod = p * e2[:, None, :]
        lo = jnp.sum(prod[:, :, :ENT_DIM], axis=2)
        hi = jnp.sum(prod[:, :, ENT_DIM:], axis=2)
        return jnp.stack([lo, hi], axis=2).reshape(NB, ENT_DIM)

    ppv = pp_scr[...]
    pnv = pn_scr[...]
    ph = matvec(ppv, hp)
    pt = matvec(ppv, tp)
    nh = matvec(pnv, hn)
    nt = matvec(pnv, tn)
    ph_ref[...] = ph
    pt_ref[...] = pt
    nh_ref[...] = nh
    nt_ref[...] = nt
    pos_ref[...] = jnp.sum(jnp.abs(ph + rp - pt), axis=1)
    neg_ref[...] = jnp.sum(jnp.abs(nh + rn - nt), axis=1)


def _tc_compute(pos_r, neg_r, proj3, ent_g, rel_g, par6):
    b = pos_r.shape[0]
    grid = (b // NB,)
    nblk = b // NB
    ent_specs = [
        pl.BlockSpec((NB, 2 * ENT_DIM),
                     functools.partial(lambda q, i, *_: (q * nblk + i, 0), q))
        for q in range(4)
    ]
    rel_specs = [
        pl.BlockSpec((NB, 2 * ENT_DIM),
                     functools.partial(lambda h, i, *_: (h * nblk + i, 0), h))
        for h in range(2)
    ]
    vec_spec = pl.BlockSpec((NB, ENT_DIM), lambda i, *_: (i, 0))
    out_shapes = (
        jax.ShapeDtypeStruct((b, ENT_DIM), jnp.float32),
        jax.ShapeDtypeStruct((b, ENT_DIM), jnp.float32),
        jax.ShapeDtypeStruct((b, ENT_DIM), jnp.float32),
        jax.ShapeDtypeStruct((b, ENT_DIM), jnp.float32),
        jax.ShapeDtypeStruct((b,), jnp.float32),
        jax.ShapeDtypeStruct((b,), jnp.float32),
    )
    return pl.pallas_call(
        _tc_body,
        grid_spec=pltpu.PrefetchScalarGridSpec(
            num_scalar_prefetch=2,
            grid=grid,
            in_specs=[
                pl.BlockSpec((proj3.shape[0], ENT_DIM // 2, 2 * ENT_DIM),
                             lambda i, *_: (0, 0, 0)),
                *ent_specs,
                *rel_specs,
                pl.BlockSpec((1, 6, NB), lambda i, *_: (i, 0, 0)),
            ],
            out_specs=[
                vec_spec, vec_spec, vec_spec, vec_spec,
                pl.BlockSpec((NB,), lambda i, *_: (i,)),
                pl.BlockSpec((NB,), lambda i, *_: (i,)),
            ],
            scratch_shapes=[
                pltpu.VMEM((NB, ENT_DIM // 2, 2 * ENT_DIM), jnp.float32),
                pltpu.VMEM((NB, ENT_DIM // 2, 2 * ENT_DIM), jnp.float32),
            ],
        ),
        out_shape=out_shapes,
    )(pos_r, neg_r, proj3, ent_g, ent_g, ent_g, ent_g, rel_g, rel_g, par6)


def kernel(pos_h, pos_t, pos_r, neg_h, neg_t, neg_r, ent_w, rel_w, proj_w):
    b = pos_h.shape[0]
    ent_idx = jnp.concatenate(
        [pos_h, pos_t, neg_h, neg_t]).astype(jnp.int32)
    rel_idx = jnp.concatenate([pos_r, neg_r]).astype(jnp.int32)
    ent2 = ent_w.reshape(ent_w.shape[0] // 2, 2 * ENT_DIM)
    rel2 = rel_w.reshape(rel_w.shape[0] // 2, 2 * ENT_DIM)
    ent_g = _sc_gather(ent2, ent_idx // 2)
    rel_g = _sc_gather(rel2, rel_idx // 2)
    all_idx = jnp.concatenate([ent_idx, rel_idx])
    par6 = (all_idx & 1).astype(jnp.float32).reshape(
        6, b // NB, NB).transpose(1, 0, 2)
    proj3 = proj_w.reshape(proj_w.shape[0], ENT_DIM // 2, 2 * ENT_DIM)
    ph, pt, nh, nt, pos, neg = _tc_compute(
        pos_r.astype(jnp.int32), neg_r.astype(jnp.int32),
        proj3, ent_g, rel_g, par6)
    return (pos, neg, ph, pt, nh, nt)


# speedup vs baseline: 1.0191x; 1.0084x over previous
import functools

import jax
import jax.numpy as jnp
from jax import lax
from jax.experimental import pallas as pl
from jax.experimental.pallas import tpu as pltpu
from jax.experimental.pallas import tpu_sc as plsc

ENT_DIM = 64
NB = 128


def _sc_gather(table2, idx):
    n = idx.shape[0]
    NW = 32
    CHUNK = n // NW
    mesh = plsc.VectorSubcoreMesh(core_axis_name="c", subcore_axis_name="s")

    @functools.partial(
        pl.kernel,
        mesh=mesh,
        out_type=jax.ShapeDtypeStruct((n, 2 * ENT_DIM), jnp.float32),
        scratch_types=[
            pltpu.VMEM((CHUNK,), jnp.int32),
            pltpu.VMEM((CHUNK, 2 * ENT_DIM), jnp.float32),
            pltpu.SemaphoreType.DMA,
        ],
    )
    def k(tab_hbm, idx_hbm, out_hbm, idx_v, rows_v, sem):
        wid = lax.axis_index("s") * 2 + lax.axis_index("c")
        base = wid * CHUNK
        pltpu.sync_copy(idx_hbm.at[pl.ds(base, CHUNK)], idx_v)
        pltpu.async_copy(tab_hbm.at[idx_v], rows_v, sem).wait()
        pltpu.sync_copy(rows_v, out_hbm.at[pl.ds(base, CHUNK)])

    return k(table2, idx)


def _sel_half(x2, par):
    return x2[:, :ENT_DIM] * (1.0 - par) + x2[:, ENT_DIM:] * par


def _tc_body(rpos_s, rneg_s, proj_ref,
             hp_ref, tp_ref, hn_ref, tn_ref, rp_ref, rn_ref, par_ref,
             ph_ref, pt_ref, nh_ref, nt_ref, pos_ref, neg_ref,
             pp_scr, pn_scr):
    i = pl.program_id(0)

    def gather_body(n, carry):
        pp_scr[n] = proj_ref[rpos_s[i * NB + n]]
        pn_scr[n] = proj_ref[rneg_s[i * NB + n]]
        return carry

    lax.fori_loop(0, NB, gather_body, 0)

    par = par_ref[0]
    lane_lt = (lax.broadcasted_iota(jnp.int32, (1, 2 * ENT_DIM), 1)
               < ENT_DIM).astype(jnp.float32)

    def _sel_both(x2, p):
        rot = jnp.roll(x2, ENT_DIM, axis=1)
        a = lane_lt * (1.0 - p) + (1.0 - lane_lt) * p
        return x2 * a + rot * (1.0 - a)

    hp = _sel_both(hp_ref[...], par[0][:, None])
    tp = _sel_both(tp_ref[...], par[1][:, None])
    hn = _sel_both(hn_ref[...], par[2][:, None])
    tn = _sel_both(tn_ref[...], par[3][:, None])
    rp = _sel_half(rp_ref[...], par[4][:, None])
    rn = _sel_half(rn_ref[...], par[5][:, None])

    def matvec(p, e2):
        prod = p * e2[:, None, :]
        lo = jnp.sum(prod[:, :, :ENT_DIM], axis=2)
        hi = jnp.sum(prod[:, :, ENT_DIM:], axis=2)
        return jnp.stack([lo, hi], axis=2).reshape(NB, ENT_DIM)

    ppv = pp_scr[...]
    pnv = pn_scr[...]
    ph = matvec(ppv, hp)
    pt = matvec(ppv, tp)
    nh = matvec(pnv, hn)
    nt = matvec(pnv, tn)
    ph_ref[...] = ph
    pt_ref[...] = pt
    nh_ref[...] = nh
    nt_ref[...] = nt
    pos_ref[...] = jnp.sum(jnp.abs(ph + rp - pt), axis=1)
    neg_ref[...] = jnp.sum(jnp.abs(nh + rn - nt), axis=1)


def _tc_compute(pos_r, neg_r, proj3, ent_g, rel_g, par6):
    b = pos_r.shape[0]
    grid = (b // NB,)
    nblk = b // NB
    ent_specs = [
        pl.BlockSpec((NB, 2 * ENT_DIM),
                     functools.partial(lambda q, i, *_: (q * nblk + i, 0), q))
        for q in range(4)
    ]
    rel_specs = [
        pl.BlockSpec((NB, 2 * ENT_DIM),
                     functools.partial(lambda h, i, *_: (h * nblk + i, 0), h))
        for h in range(2)
    ]
    vec_spec = pl.BlockSpec((NB, ENT_DIM), lambda i, *_: (i, 0))
    out_shapes = (
        jax.ShapeDtypeStruct((b, ENT_DIM), jnp.float32),
        jax.ShapeDtypeStruct((b, ENT_DIM), jnp.float32),
        jax.ShapeDtypeStruct((b, ENT_DIM), jnp.float32),
        jax.ShapeDtypeStruct((b, ENT_DIM), jnp.float32),
        jax.ShapeDtypeStruct((b,), jnp.float32),
        jax.ShapeDtypeStruct((b,), jnp.float32),
    )
    return pl.pallas_call(
        _tc_body,
        grid_spec=pltpu.PrefetchScalarGridSpec(
            num_scalar_prefetch=2,
            grid=grid,
            in_specs=[
                pl.BlockSpec((proj3.shape[0], ENT_DIM // 2, 2 * ENT_DIM),
                             lambda i, *_: (0, 0, 0)),
                *ent_specs,
                *rel_specs,
                pl.BlockSpec((1, 6, NB), lambda i, *_: (i, 0, 0)),
            ],
            out_specs=[
                vec_spec, vec_spec, vec_spec, vec_spec,
                pl.BlockSpec((NB,), lambda i, *_: (i,)),
                pl.BlockSpec((NB,), lambda i, *_: (i,)),
            ],
            scratch_shapes=[
                pltpu.VMEM((NB, ENT_DIM // 2, 2 * ENT_DIM), jnp.float32),
                pltpu.VMEM((NB, ENT_DIM // 2, 2 * ENT_DIM), jnp.float32),
            ],
        ),
        out_shape=out_shapes,
        compiler_params=pltpu.CompilerParams(
            dimension_semantics=("parallel",)),
    )(pos_r, neg_r, proj3, ent_g, ent_g, ent_g, ent_g, rel_g, rel_g, par6)


def kernel(pos_h, pos_t, pos_r, neg_h, neg_t, neg_r, ent_w, rel_w, proj_w):
    b = pos_h.shape[0]
    ent_idx = jnp.concatenate(
        [pos_h, pos_t, neg_h, neg_t]).astype(jnp.int32)
    rel_idx = jnp.concatenate([pos_r, neg_r]).astype(jnp.int32)
    ent2 = ent_w.reshape(ent_w.shape[0] // 2, 2 * ENT_DIM)
    rel2 = rel_w.reshape(rel_w.shape[0] // 2, 2 * ENT_DIM)
    ent_g = _sc_gather(ent2, ent_idx // 2)
    rel_g = _sc_gather(rel2, rel_idx // 2)
    all_idx = jnp.concatenate([ent_idx, rel_idx])
    par6 = (all_idx & 1).astype(jnp.float32).reshape(
        6, b // NB, NB).transpose(1, 0, 2)
    proj3 = proj_w.reshape(proj_w.shape[0], ENT_DIM // 2, 2 * ENT_DIM)
    ph, pt, nh, nt, pos, neg = _tc_compute(
        pos_r.astype(jnp.int32), neg_r.astype(jnp.int32),
        proj3, ent_g, rel_g, par6)
    return (pos, neg, ph, pt, nh, nt)
